# baseline (speedup 1.0000x reference)
import functools

import jax
import jax.numpy as jnp
from jax import lax
from jax.experimental import pallas as pl
from jax.experimental.pallas import tpu as pltpu
from jax.experimental.pallas import tpu_sc as plsc

B = 4096
L = 200
D = 64
NCAT_P = 1008
NSMALL = 16
NW = 32
RPW = B // NW
NG = RPW // 16


def _sc_body(item_seq, cat_seq, inter_seq, dur_seq, item_id, video_cat, age,
             gender, item_emb, cat_emb, age_emb, gender_emb,
             o_sum_item, o_cat_cnt, o_inter_cnt, o_dur_cnt, o_ei, o_ec,
             o_eage, o_egen,
             ia0, ib0, ia1, ib1, rows0, rows1, acc, slab_it, slab_ct,
             slab_in, slab_du, hist_c, hist_i, hist_d, buf32,
             sem0, sem1):
  nc = 2
  wid = lax.axis_index("s") * nc + lax.axis_index("c")
  base = wid * RPW

  zf = jnp.zeros((16,), jnp.float32)
  ones16 = jnp.ones((16,), jnp.float32)
  row_iota = lax.iota(jnp.int32, 16)

  def single_gather(idx_src, table, out, dstbuf):
    pltpu.sync_copy(idx_src.at[pl.ds(base, RPW)], ia0)
    pltpu.async_copy(table.at[ia0], dstbuf, sem0).wait()
    pltpu.sync_copy(dstbuf, out.at[pl.ds(base, RPW)])

  single_gather(item_id, item_emb, o_ei, rows0.at[pl.ds(0, RPW)])
  single_gather(video_cat, cat_emb, o_ec, rows0.at[pl.ds(0, RPW)])
  single_gather(age, age_emb, o_eage, buf32)
  single_gather(gender, gender_emb, o_egen, buf32)

  def zero_hist(r, _):
    def zcol(j, _):
      hist_c[r, pl.ds(j * 16, 16)] = zf
      return None
    lax.fori_loop(0, NCAT_P // 16, zcol, None)
    hist_i[r, :] = zf
    hist_d[r, :] = zf
    return None
  lax.fori_loop(0, 16, zero_hist, None)

  row_off = row_iota * L
  def group(g, _):
    r0 = base + g * 16
    pltpu.sync_copy(item_seq.at[pl.ds(r0 * L, 16 * L)], slab_it)
    pltpu.sync_copy(cat_seq.at[pl.ds(r0 * L, 16 * L)], slab_ct)
    pltpu.sync_copy(inter_seq.at[pl.ds(r0 * L, 16 * L)], slab_in)
    pltpu.sync_copy(dur_seq.at[pl.ds(r0 * L, 16 * L)], slab_du)

    def tok(t, _):
      tv = row_off + t
      it_v = plsc.load_gather(slab_it, [tv])
      m = it_v > 0
      ct_v = plsc.load_gather(slab_ct, [tv])
      in_v = plsc.load_gather(slab_in, [tv])
      du_v = plsc.load_gather(slab_du, [tv])
      plsc.addupdate_scatter(hist_c, [row_iota, ct_v], ones16, mask=m)
      plsc.addupdate_scatter(hist_i, [row_iota, in_v], ones16, mask=m)
      plsc.addupdate_scatter(hist_d, [row_iota, du_v], ones16, mask=m)
      return None
    lax.fori_loop(0, L, tok, None)

    pltpu.sync_copy(hist_c, o_cat_cnt.at[pl.ds(r0, 16)])
    pltpu.sync_copy(hist_i, o_inter_cnt.at[pl.ds(r0, 16)])
    pltpu.sync_copy(hist_d, o_dur_cnt.at[pl.ds(r0, 16)])
    lax.fori_loop(0, 16, zero_hist, None)
    return None
  lax.fori_loop(0, NG, group, None)

  def stage_idx(r, ia, ib):
    pltpu.sync_copy(item_seq.at[pl.ds(r * L, 128)], ia)
    pltpu.sync_copy(item_seq.at[pl.ds(r * L + 128, 72)], ib)

  def start_gather(ia, ib, buf, sem):
    pltpu.async_copy(item_emb.at[ia], buf.at[pl.ds(0, 128)], sem)
    pltpu.async_copy(item_emb.at[ib], buf.at[pl.ds(128, 72)], sem)

  def wait_gather(ia, ib, buf, sem):
    pltpu.make_async_copy(item_emb.at[ia], buf.at[pl.ds(0, 128)], sem).wait()
    pltpu.make_async_copy(item_emb.at[ib], buf.at[pl.ds(128, 72)], sem).wait()

  def acc_row(buf, r):
    def tokd(j, carry):
      a0, a1, a2, a3 = carry
      t = j * 2
      a0 = a0 + buf[t, pl.ds(0, 16)] + buf[t + 1, pl.ds(0, 16)]
      a1 = a1 + buf[t, pl.ds(16, 16)] + buf[t + 1, pl.ds(16, 16)]
      a2 = a2 + buf[t, pl.ds(32, 16)] + buf[t + 1, pl.ds(32, 16)]
      a3 = a3 + buf[t, pl.ds(48, 16)] + buf[t + 1, pl.ds(48, 16)]
      return (a0, a1, a2, a3)
    a0, a1, a2, a3 = lax.fori_loop(0, L // 2, tokd, (zf, zf, zf, zf))
    acc[r, pl.ds(0, 16)] = a0
    acc[r, pl.ds(16, 16)] = a1
    acc[r, pl.ds(32, 16)] = a2
    acc[r, pl.ds(48, 16)] = a3

  stage_idx(base, ia0, ib0)
  start_gather(ia0, ib0, rows0, sem0)

  def pair(j, _):
    r0 = base + j * 2
    stage_idx(r0 + 1, ia1, ib1)
    start_gather(ia1, ib1, rows1, sem1)
    wait_gather(ia0, ib0, rows0, sem0)
    acc_row(rows0, j * 2)

    @pl.when(j + 1 < RPW // 2)
    def _():
      stage_idx(r0 + 2, ia0, ib0)
      start_gather(ia0, ib0, rows0, sem0)
    wait_gather(ia1, ib1, rows1, sem1)
    acc_row(rows1, j * 2 + 1)
    return None
  lax.fori_loop(0, RPW // 2, pair, None)

  pltpu.sync_copy(acc, o_sum_item.at[pl.ds(base, RPW)])


def _sc_call(item_seq, cat_seq, inter_seq, dur_seq, item_id, video_cat, age,
             gender, item_emb, cat_emb, age_emb, gender_emb):
  f32 = jnp.float32
  mesh = plsc.VectorSubcoreMesh(core_axis_name="c", subcore_axis_name="s")
  out_type = [
      jax.ShapeDtypeStruct((B, D), f32),
      jax.ShapeDtypeStruct((B, NCAT_P), f32),
      jax.ShapeDtypeStruct((B, NSMALL), f32),
      jax.ShapeDtypeStruct((B, NSMALL), f32),
      jax.ShapeDtypeStruct((B, D), f32),
      jax.ShapeDtypeStruct((B, D), f32),
      jax.ShapeDtypeStruct((B, D // 2), f32),
      jax.ShapeDtypeStruct((B, D // 2), f32),
  ]
  scratch = [
      pltpu.VMEM((128,), jnp.int32),
      pltpu.VMEM((72,), jnp.int32),
      pltpu.VMEM((128,), jnp.int32),
      pltpu.VMEM((72,), jnp.int32),
      pltpu.VMEM((L, D), f32),
      pltpu.VMEM((L, D), f32),
      pltpu.VMEM((RPW, D), f32),
      pltpu.VMEM((16 * L,), jnp.int32),
      pltpu.VMEM((16 * L,), jnp.int32),
      pltpu.VMEM((16 * L,), jnp.int32),
      pltpu.VMEM((16 * L,), jnp.int32),
      pltpu.VMEM((16, NCAT_P), f32),
      pltpu.VMEM((16, NSMALL), f32),
      pltpu.VMEM((16, NSMALL), f32),
      pltpu.VMEM((RPW, D // 2), f32),
      pltpu.SemaphoreType.DMA,
      pltpu.SemaphoreType.DMA,
  ]
  fn = pl.kernel(_sc_body, out_type=out_type, mesh=mesh,
                 scratch_types=scratch,
                 compiler_params=pltpu.CompilerParams(
                     use_tc_tiling_on_sc=False,
                     needs_layout_passes=False))
  return fn(item_seq, cat_seq, inter_seq, dur_seq, item_id, video_cat, age,
            gender, item_emb, cat_emb, age_emb, gender_emb)


def _tc_cat_body(cat_cnt, cat_t, o_catsum, o_c):
  cc = cat_cnt[...]
  o_catsum[...] = jnp.dot(cc, cat_t[...], precision=jax.lax.Precision.HIGHEST)
  o_c[...] = jnp.sum(cc, axis=1, keepdims=True)


def _tc_body(sum_item, catsum, c_in, inter_cnt, dur_cnt, ei, ec, eage, egen,
             ua, pop, e0, inter_t, dur_t,
             w_it, w_ct, w_in, w_du, w_ua, w_ag, w_ge, ub0, ug0, ube0,
             uW1, ub1, ug1, ube1, uW2, ub2,
             iw_ei, iw_ec, iw_pop, ib0, ig0, ibe0,
             iW1, ib1, ig1, ibe1, iW2, ib2,
             o_u, o_i):
  hi = jax.lax.Precision.HIGHEST

  def dot(a, b):
    return jnp.dot(a, b, precision=hi)

  def bn(x, g, b):
    m = jnp.mean(x, axis=0, keepdims=True)
    v = jnp.mean((x - m) * (x - m), axis=0, keepdims=True)
    return g[...] * (x - m) / jnp.sqrt(v + 1e-5) + b[...]

  def l2n(x):
    n = jnp.sqrt(jnp.sum(x * x, axis=1, keepdims=True))
    return x / jnp.maximum(n, 1e-12)

  c = c_in[...]
  denom = jnp.maximum(c, 1e-9)
  n0 = jnp.float32(L) - c
  p_item = (sum_item[...] - n0 * e0[...]) / denom
  p_cat = catsum[...] / denom
  p_inter = dot(inter_cnt[...], inter_t[...]) / denom
  p_dur = dot(dur_cnt[...], dur_t[...]) / denom

  h = (dot(p_item, w_it[...]) + dot(p_cat, w_ct[...]) +
       dot(p_inter, w_in[...]) + dot(p_dur, w_du[...]) +
       ua[...] * w_ua[...] + dot(eage[...], w_ag[...]) +
       dot(egen[...], w_ge[...]) + ub0[...])
  h = jnp.maximum(bn(h, ug0[...], ube0[...]), 0.0)
  h = jnp.maximum(bn(dot(h, uW1[...]) + ub1[...], ug1[...], ube1[...]), 0.0)
  u = dot(h, uW2[...]) + ub2[...]

  h = (dot(ei[...], iw_ei[...]) + dot(ec[...], iw_ec[...]) +
       pop[...] * iw_pop[...] + ib0[...])
  h = jnp.maximum(bn(h, ig0[...], ibe0[...]), 0.0)
  h = jnp.maximum(bn(dot(h, iW1[...]) + ib1[...], ig1[...], ibe1[...]), 0.0)
  iv = dot(h, iW2[...]) + ib2[...]

  o_u[...] = l2n(u)
  o_i[...] = l2n(iv)


def kernel(item_id_seq, video_category_seq, inter_type_seq,
           duration_bucket_seq, user_activity_norm, age, gender, item_id,
           video_category, item_pop_norm, item_emb, cat_emb, inter_emb,
           dur_emb, age_emb, gender_emb, uW0, ub0, ug0, ube0, uW1, ub1, ug1,
           ube1, uW2, ub2, iW0, ib0, ig0, ibe0, iW1, ib1, ig1, ibe1, iW2,
           ib2):
  i32 = jnp.int32
  f32 = jnp.float32
  item_seq = item_id_seq.astype(i32).reshape(-1)
  cat_seq = video_category_seq.astype(i32).reshape(-1)
  inter_seq = inter_type_seq.astype(i32).reshape(-1)
  dur_seq = duration_bucket_seq.astype(i32).reshape(-1)

  (sum_item, cat_cnt, inter_cnt, dur_cnt, ei, ec, eage, egen) = _sc_call(
      item_seq, cat_seq, inter_seq, dur_seq, item_id.astype(i32),
      video_category.astype(i32), age.astype(i32), gender.astype(i32),
      item_emb, cat_emb, age_emb, gender_emb)

  e0 = lax.slice(item_emb, (0, 0), (1, D))
  cat_t = jnp.pad(cat_emb, ((0, NCAT_P - cat_emb.shape[0]), (0, 0)))
  inter_t = jnp.pad(inter_emb, ((0, NSMALL - inter_emb.shape[0]), (0, 0)))
  dur_t = jnp.pad(dur_emb, ((0, NSMALL - dur_emb.shape[0]), (0, 0)))

  w_it = lax.slice(uW0, (0, 0), (D, 128))
  w_ct = lax.slice(uW0, (D, 0), (2 * D, 128))
  w_in = lax.slice(uW0, (2 * D, 0), (3 * D, 128))
  w_du = lax.slice(uW0, (3 * D, 0), (4 * D, 128))
  w_ua = lax.slice(uW0, (4 * D, 0), (4 * D + 1, 128))
  w_ag = lax.slice(uW0, (4 * D + 1, 0), (4 * D + 1 + D // 2, 128))
  w_ge = lax.slice(uW0, (4 * D + 1 + D // 2, 0), (4 * D + 1 + D, 128))
  iw_ei = lax.slice(iW0, (0, 0), (D, 128))
  iw_ec = lax.slice(iW0, (D, 0), (2 * D, 128))
  iw_pop = lax.slice(iW0, (2 * D, 0), (2 * D + 1, 128))

  def r2(v):
    return v.reshape(1, -1)

  nblk = 8
  rows = B // nblk
  catsum, c = pl.pallas_call(
      _tc_cat_body,
      grid=(nblk,),
      in_specs=[pl.BlockSpec((rows, NCAT_P), lambda i: (i, 0)),
                pl.BlockSpec((NCAT_P, D), lambda i: (0, 0))],
      out_specs=[pl.BlockSpec((rows, D), lambda i: (i, 0)),
                 pl.BlockSpec((rows, 1), lambda i: (i, 0))],
      out_shape=[jax.ShapeDtypeStruct((B, D), f32),
                 jax.ShapeDtypeStruct((B, 1), f32)])(cat_cnt, cat_t)

  out_shape = (jax.ShapeDtypeStruct((B, D), f32),
               jax.ShapeDtypeStruct((B, D), f32))
  return pl.pallas_call(_tc_body, out_shape=out_shape)(
      sum_item, catsum, c, inter_cnt, dur_cnt, ei, ec, eage, egen,
      user_activity_norm, item_pop_norm, e0, inter_t, dur_t,
      w_it, w_ct, w_in, w_du, w_ua, w_ag, w_ge,
      r2(ub0), r2(ug0), r2(ube0), uW1, r2(ub1), r2(ug1), r2(ube1),
      uW2, r2(ub2),
      iw_ei, iw_ec, iw_pop, r2(ib0), r2(ig0), r2(ibe0),
      iW1, r2(ib1), r2(ig1), r2(ibe1), iW2, r2(ib2))


# speedup vs baseline: 1.0582x; 1.0582x over previous
import functools

import jax
import jax.numpy as jnp
from jax import lax
from jax.experimental import pallas as pl
from jax.experimental.pallas import tpu as pltpu
from jax.experimental.pallas import tpu_sc as plsc

B = 4096
L = 200
D = 64
NCAT_P = 1008
NSMALL = 16
NW = 32
RPW = B // NW
NG = RPW // 16


def _sc_body(item_seq, cat_seq, inter_seq, dur_seq, item_id, video_cat, age,
             gender, item_emb, cat_emb, age_emb, gender_emb,
             o_sum_item, o_cat_cnt, o_inter_cnt, o_dur_cnt, o_ei, o_ec,
             o_eage, o_egen,
             ia0, ib0, ia1, ib1, rows0, rows1, acc, slab_it, slab_ct,
             slab_in, slab_du, hist_c, hist_i, hist_d,
             sem0, sem1):
  nc = 2
  wid = lax.axis_index("s") * nc + lax.axis_index("c")
  base = wid * RPW

  zf = jnp.zeros((16,), jnp.float32)
  ones16 = jnp.ones((16,), jnp.float32)
  row_iota = lax.iota(jnp.int32, 16)

  def single_gather(idx_src, table, out, dstbuf):
    pltpu.sync_copy(idx_src.at[pl.ds(base, RPW)], ia0)
    pltpu.async_copy(table.at[ia0], dstbuf, sem0).wait()
    pltpu.sync_copy(dstbuf, out.at[pl.ds(base, RPW)])

  single_gather(item_id, item_emb, o_ei, rows0.at[pl.ds(0, RPW)])
  single_gather(video_cat, cat_emb, o_ec, rows0.at[pl.ds(0, RPW)])
  single_gather(age, age_emb, o_eage, rows0.at[pl.ds(0, RPW)])
  single_gather(gender, gender_emb, o_egen, rows0.at[pl.ds(0, RPW)])

  def zero_hist(r, _):
    def zcol(j, _):
      hist_c[r, pl.ds(j * 16, 16)] = zf
      return None
    lax.fori_loop(0, NCAT_P // 16, zcol, None)
    hist_i[r, :] = zf
    hist_d[r, :] = zf
    return None
  lax.fori_loop(0, 16, zero_hist, None)

  row_off = row_iota * L
  def group(g, _):
    r0 = base + g * 16
    pltpu.sync_copy(item_seq.at[pl.ds(r0 * L, 16 * L)], slab_it)
    pltpu.sync_copy(cat_seq.at[pl.ds(r0 * L, 16 * L)], slab_ct)
    pltpu.sync_copy(inter_seq.at[pl.ds(r0 * L, 16 * L)], slab_in)
    pltpu.sync_copy(dur_seq.at[pl.ds(r0 * L, 16 * L)], slab_du)

    def tok(t, _):
      tv = row_off + t
      it_v = plsc.load_gather(slab_it, [tv])
      m = it_v > 0
      ct_v = plsc.load_gather(slab_ct, [tv])
      in_v = plsc.load_gather(slab_in, [tv])
      du_v = plsc.load_gather(slab_du, [tv])
      plsc.addupdate_scatter(hist_c, [row_iota, ct_v], ones16, mask=m)
      plsc.addupdate_scatter(hist_i, [row_iota, in_v], ones16, mask=m)
      plsc.addupdate_scatter(hist_d, [row_iota, du_v], ones16, mask=m)
      return None
    lax.fori_loop(0, L, tok, None)

    pltpu.sync_copy(hist_c, o_cat_cnt.at[pl.ds(r0, 16)])
    pltpu.sync_copy(hist_i, o_inter_cnt.at[pl.ds(r0, 16)])
    pltpu.sync_copy(hist_d, o_dur_cnt.at[pl.ds(r0, 16)])
    lax.fori_loop(0, 16, zero_hist, None)
    return None
  lax.fori_loop(0, NG, group, None)

  def stage_idx(r, ia, ib):
    pltpu.sync_copy(item_seq.at[pl.ds(r * L, 128)], ia)
    pltpu.sync_copy(item_seq.at[pl.ds(r * L + 128, 72)], ib)

  def start_gather(ia, ib, buf, sem):
    pltpu.async_copy(item_emb.at[ia], buf.at[pl.ds(0, 128)], sem)
    pltpu.async_copy(item_emb.at[ib], buf.at[pl.ds(128, 72)], sem)

  def wait_gather(ia, ib, buf, sem):
    pltpu.make_async_copy(item_emb.at[ia], buf.at[pl.ds(0, 128)], sem).wait()
    pltpu.make_async_copy(item_emb.at[ib], buf.at[pl.ds(128, 72)], sem).wait()

  def acc_row(buf, r):
    def tokd(j, carry):
      a0, a1, a2, a3 = carry
      t = j * 2
      a0 = a0 + buf[t, pl.ds(0, 16)] + buf[t + 1, pl.ds(0, 16)]
      a1 = a1 + buf[t, pl.ds(16, 16)] + buf[t + 1, pl.ds(16, 16)]
      a2 = a2 + buf[t, pl.ds(32, 16)] + buf[t + 1, pl.ds(32, 16)]
      a3 = a3 + buf[t, pl.ds(48, 16)] + buf[t + 1, pl.ds(48, 16)]
      return (a0, a1, a2, a3)
    a0, a1, a2, a3 = lax.fori_loop(0, L // 2, tokd, (zf, zf, zf, zf))
    acc[r, pl.ds(0, 16)] = a0
    acc[r, pl.ds(16, 16)] = a1
    acc[r, pl.ds(32, 16)] = a2
    acc[r, pl.ds(48, 16)] = a3

  stage_idx(base, ia0, ib0)
  start_gather(ia0, ib0, rows0, sem0)

  def pair(j, _):
    r0 = base + j * 2
    stage_idx(r0 + 1, ia1, ib1)
    start_gather(ia1, ib1, rows1, sem1)
    wait_gather(ia0, ib0, rows0, sem0)
    acc_row(rows0, j * 2)

    @pl.when(j + 1 < RPW // 2)
    def _():
      stage_idx(r0 + 2, ia0, ib0)
      start_gather(ia0, ib0, rows0, sem0)
    wait_gather(ia1, ib1, rows1, sem1)
    acc_row(rows1, j * 2 + 1)
    return None
  lax.fori_loop(0, RPW // 2, pair, None)

  pltpu.sync_copy(acc, o_sum_item.at[pl.ds(base, RPW)])


def _sc_call(item_seq, cat_seq, inter_seq, dur_seq, item_id, video_cat, age,
             gender, item_emb, cat_emb, age_emb, gender_emb):
  f32 = jnp.float32
  mesh = plsc.VectorSubcoreMesh(core_axis_name="c", subcore_axis_name="s")
  out_type = [
      jax.ShapeDtypeStruct((B, D), f32),
      jax.ShapeDtypeStruct((B, NCAT_P), f32),
      jax.ShapeDtypeStruct((B, NSMALL), f32),
      jax.ShapeDtypeStruct((B, NSMALL), f32),
      jax.ShapeDtypeStruct((B, 128), f32),
      jax.ShapeDtypeStruct((B, 128), f32),
      jax.ShapeDtypeStruct((B, 128), f32),
      jax.ShapeDtypeStruct((B, 128), f32),
  ]
  scratch = [
      pltpu.VMEM((128,), jnp.int32),
      pltpu.VMEM((72,), jnp.int32),
      pltpu.VMEM((128,), jnp.int32),
      pltpu.VMEM((72,), jnp.int32),
      pltpu.VMEM((L, 128), f32),
      pltpu.VMEM((L, 128), f32),
      pltpu.VMEM((RPW, D), f32),
      pltpu.VMEM((16 * L,), jnp.int32),
      pltpu.VMEM((16 * L,), jnp.int32),
      pltpu.VMEM((16 * L,), jnp.int32),
      pltpu.VMEM((16 * L,), jnp.int32),
      pltpu.VMEM((16, NCAT_P), f32),
      pltpu.VMEM((16, NSMALL), f32),
      pltpu.VMEM((16, NSMALL), f32),
      pltpu.SemaphoreType.DMA,
      pltpu.SemaphoreType.DMA,
  ]
  fn = pl.kernel(_sc_body, out_type=out_type, mesh=mesh,
                 scratch_types=scratch,
                 compiler_params=pltpu.CompilerParams(
                     use_tc_tiling_on_sc=True,
                     needs_layout_passes=False))
  return fn(item_seq, cat_seq, inter_seq, dur_seq, item_id, video_cat, age,
            gender, item_emb, cat_emb, age_emb, gender_emb)


def _tc_cat_body(cat_cnt, cat_t, o_catsum, o_c):
  cc = cat_cnt[...]
  o_catsum[...] = jnp.dot(cc, cat_t[...], precision=jax.lax.Precision.HIGHEST)
  o_c[...] = jnp.sum(cc, axis=1, keepdims=True)


def _tc_body(sum_item, catsum, c_in, inter_cnt, dur_cnt, ei, ec, eage, egen,
             ua, pop, e0, inter_t, dur_t,
             w_it, w_ct, w_in, w_du, w_ua, w_ag, w_ge, ub0, ug0, ube0,
             uW1, ub1, ug1, ube1, uW2, ub2,
             iw_ei, iw_ec, iw_pop, ib0, ig0, ibe0,
             iW1, ib1, ig1, ibe1, iW2, ib2,
             o_u, o_i):
  hi = jax.lax.Precision.HIGHEST

  def dot(a, b):
    return jnp.dot(a, b, precision=hi)

  def bn(x, g, b):
    m = jnp.mean(x, axis=0, keepdims=True)
    v = jnp.mean((x - m) * (x - m), axis=0, keepdims=True)
    return g[...] * (x - m) / jnp.sqrt(v + 1e-5) + b[...]

  def l2n(x):
    n = jnp.sqrt(jnp.sum(x * x, axis=1, keepdims=True))
    return x / jnp.maximum(n, 1e-12)

  c = c_in[...]
  denom = jnp.maximum(c, 1e-9)
  n0 = jnp.float32(L) - c
  p_item = (sum_item[...] - n0 * e0[...]) / denom
  p_cat = catsum[...] / denom
  p_inter = dot(inter_cnt[...], inter_t[...]) / denom
  p_dur = dot(dur_cnt[...], dur_t[...]) / denom

  h = (dot(p_item, w_it[...]) + dot(p_cat, w_ct[...]) +
       dot(p_inter, w_in[...]) + dot(p_dur, w_du[...]) +
       ua[...] * w_ua[...] + dot(eage[...], w_ag[...]) +
       dot(egen[...], w_ge[...]) + ub0[...])
  h = jnp.maximum(bn(h, ug0[...], ube0[...]), 0.0)
  h = jnp.maximum(bn(dot(h, uW1[...]) + ub1[...], ug1[...], ube1[...]), 0.0)
  u = dot(h, uW2[...]) + ub2[...]

  h = (dot(ei[...], iw_ei[...]) + dot(ec[...], iw_ec[...]) +
       pop[...] * iw_pop[...] + ib0[...])
  h = jnp.maximum(bn(h, ig0[...], ibe0[...]), 0.0)
  h = jnp.maximum(bn(dot(h, iW1[...]) + ib1[...], ig1[...], ibe1[...]), 0.0)
  iv = dot(h, iW2[...]) + ib2[...]

  o_u[...] = l2n(u)
  o_i[...] = l2n(iv)


def kernel(item_id_seq, video_category_seq, inter_type_seq,
           duration_bucket_seq, user_activity_norm, age, gender, item_id,
           video_category, item_pop_norm, item_emb, cat_emb, inter_emb,
           dur_emb, age_emb, gender_emb, uW0, ub0, ug0, ube0, uW1, ub1, ug1,
           ube1, uW2, ub2, iW0, ib0, ig0, ibe0, iW1, ib1, ig1, ibe1, iW2,
           ib2):
  i32 = jnp.int32
  f32 = jnp.float32
  item_seq = item_id_seq.astype(i32).reshape(-1)
  cat_seq = video_category_seq.astype(i32).reshape(-1)
  inter_seq = inter_type_seq.astype(i32).reshape(-1)
  dur_seq = duration_bucket_seq.astype(i32).reshape(-1)

  item_pad = jnp.pad(item_emb, ((0, 0), (0, 128 - D)))
  cat_pad = jnp.pad(cat_emb, ((0, 0), (0, 128 - D)))
  age_pad = jnp.pad(age_emb, ((0, 0), (0, 128 - D // 2)))
  gen_pad = jnp.pad(gender_emb, ((0, 0), (0, 128 - D // 2)))

  (sum_item, cat_cnt, inter_cnt, dur_cnt, ei128, ec128, eage128,
   egen128) = _sc_call(
      item_seq, cat_seq, inter_seq, dur_seq, item_id.astype(i32),
      video_category.astype(i32), age.astype(i32), gender.astype(i32),
      item_pad, cat_pad, age_pad, gen_pad)
  ei = lax.slice(ei128, (0, 0), (B, D))
  ec = lax.slice(ec128, (0, 0), (B, D))
  eage = lax.slice(eage128, (0, 0), (B, D // 2))
  egen = lax.slice(egen128, (0, 0), (B, D // 2))

  e0 = lax.slice(item_emb, (0, 0), (1, D))
  cat_t = jnp.pad(cat_emb, ((0, NCAT_P - cat_emb.shape[0]), (0, 0)))
  inter_t = jnp.pad(inter_emb, ((0, NSMALL - inter_emb.shape[0]), (0, 0)))
  dur_t = jnp.pad(dur_emb, ((0, NSMALL - dur_emb.shape[0]), (0, 0)))

  w_it = lax.slice(uW0, (0, 0), (D, 128))
  w_ct = lax.slice(uW0, (D, 0), (2 * D, 128))
  w_in = lax.slice(uW0, (2 * D, 0), (3 * D, 128))
  w_du = lax.slice(uW0, (3 * D, 0), (4 * D, 128))
  w_ua = lax.slice(uW0, (4 * D, 0), (4 * D + 1, 128))
  w_ag = lax.slice(uW0, (4 * D + 1, 0), (4 * D + 1 + D // 2, 128))
  w_ge = lax.slice(uW0, (4 * D + 1 + D // 2, 0), (4 * D + 1 + D, 128))
  iw_ei = lax.slice(iW0, (0, 0), (D, 128))
  iw_ec = lax.slice(iW0, (D, 0), (2 * D, 128))
  iw_pop = lax.slice(iW0, (2 * D, 0), (2 * D + 1, 128))

  def r2(v):
    return v.reshape(1, -1)

  nblk = 8
  rows = B // nblk
  catsum, c = pl.pallas_call(
      _tc_cat_body,
      grid=(nblk,),
      in_specs=[pl.BlockSpec((rows, NCAT_P), lambda i: (i, 0)),
                pl.BlockSpec((NCAT_P, D), lambda i: (0, 0))],
      out_specs=[pl.BlockSpec((rows, D), lambda i: (i, 0)),
                 pl.BlockSpec((rows, 1), lambda i: (i, 0))],
      out_shape=[jax.ShapeDtypeStruct((B, D), f32),
                 jax.ShapeDtypeStruct((B, 1), f32)])(cat_cnt, cat_t)

  out_shape = (jax.ShapeDtypeStruct((B, D), f32),
               jax.ShapeDtypeStruct((B, D), f32))
  return pl.pallas_call(_tc_body, out_shape=out_shape)(
      sum_item, catsum, c, inter_cnt, dur_cnt, ei, ec, eage, egen,
      user_activity_norm, item_pop_norm, e0, inter_t, dur_t,
      w_it, w_ct, w_in, w_du, w_ua, w_ag, w_ge,
      r2(ub0), r2(ug0), r2(ube0), uW1, r2(ub1), r2(ug1), r2(ube1),
      uW2, r2(ub2),
      iw_ei, iw_ec, iw_pop, r2(ib0), r2(ig0), r2(ibe0),
      iW1, r2(ib1), r2(ig1), r2(ibe1), iW2, r2(ib2))


# speedup vs baseline: 1.0597x; 1.0014x over previous
import functools

import jax
import jax.numpy as jnp
from jax import lax
from jax.experimental import pallas as pl
from jax.experimental.pallas import tpu as pltpu
from jax.experimental.pallas import tpu_sc as plsc

B = 4096
L = 200
D = 64
NCAT_P = 1008
NSMALL = 16
NW = 32
RPW = B // NW
NG = RPW // 16


def _sc_body(item_seq, cat_seq, inter_seq, dur_seq, item_id, video_cat, age,
             gender, item_emb, cat_emb, age_emb, gender_emb,
             o_sum_item, o_cat_cnt, o_inter_cnt, o_dur_cnt, o_ei, o_ec,
             o_eage, o_egen,
             ia0, ib0, ia1, ib1, rows0, rows1, acc, slab_it, slab_ct,
             slab_in, slab_du, hist_c, hist_i, hist_d,
             sem0, sem1):
  nc = 2
  wid = lax.axis_index("s") * nc + lax.axis_index("c")
  base = wid * RPW

  zf = jnp.zeros((16,), jnp.float32)
  ones16 = jnp.ones((16,), jnp.float32)
  row_iota = lax.iota(jnp.int32, 16)

  def single_gather(idx_src, table, out, dstbuf):
    pltpu.sync_copy(idx_src.at[pl.ds(base, RPW)], ia0)
    pltpu.async_copy(table.at[ia0], dstbuf, sem0).wait()
    pltpu.sync_copy(dstbuf, out.at[pl.ds(base, RPW)])

  single_gather(item_id, item_emb, o_ei, rows0.at[pl.ds(0, RPW)])
  single_gather(video_cat, cat_emb, o_ec, rows0.at[pl.ds(0, RPW)])
  single_gather(age, age_emb, o_eage, rows0.at[pl.ds(0, RPW)])
  single_gather(gender, gender_emb, o_egen, rows0.at[pl.ds(0, RPW)])

  def zero_hist(r, _):
    def zcol(j, _):
      hist_c[r, pl.ds(j * 16, 16)] = zf
      return None
    lax.fori_loop(0, NCAT_P // 16, zcol, None)
    hist_i[r, :] = zf
    hist_d[r, :] = zf
    return None
  lax.fori_loop(0, 16, zero_hist, None)

  row_off = row_iota * L
  def group(g, _):
    r0 = base + g * 16
    pltpu.sync_copy(item_seq.at[pl.ds(r0 * L, 16 * L)], slab_it)
    pltpu.sync_copy(cat_seq.at[pl.ds(r0 * L, 16 * L)], slab_ct)
    pltpu.sync_copy(inter_seq.at[pl.ds(r0 * L, 16 * L)], slab_in)
    pltpu.sync_copy(dur_seq.at[pl.ds(r0 * L, 16 * L)], slab_du)

    def tok(t, _):
      tv = row_off + t
      it_v = plsc.load_gather(slab_it, [tv])
      m = it_v > 0
      ct_v = plsc.load_gather(slab_ct, [tv])
      in_v = plsc.load_gather(slab_in, [tv])
      du_v = plsc.load_gather(slab_du, [tv])
      plsc.addupdate_scatter(hist_c, [row_iota, ct_v], ones16, mask=m)
      plsc.addupdate_scatter(hist_i, [row_iota, in_v], ones16, mask=m)
      plsc.addupdate_scatter(hist_d, [row_iota, du_v], ones16, mask=m)
      return None
    lax.fori_loop(0, L, tok, None)

    pltpu.sync_copy(hist_c, o_cat_cnt.at[pl.ds(r0, 16)])
    pltpu.sync_copy(hist_i, o_inter_cnt.at[pl.ds(r0, 16)])
    pltpu.sync_copy(hist_d, o_dur_cnt.at[pl.ds(r0, 16)])
    lax.fori_loop(0, 16, zero_hist, None)
    return None
  lax.fori_loop(0, NG, group, None)

  def stage_idx(r, ia, ib):
    pltpu.sync_copy(item_seq.at[pl.ds(r * L, 128)], ia)
    pltpu.sync_copy(item_seq.at[pl.ds(r * L + 128, 72)], ib)

  def start_gather(ia, ib, buf, sem):
    pltpu.async_copy(item_emb.at[ia], buf.at[pl.ds(0, 128)], sem)
    pltpu.async_copy(item_emb.at[ib], buf.at[pl.ds(128, 72)], sem)

  def wait_gather(ia, ib, buf, sem):
    pltpu.make_async_copy(item_emb.at[ia], buf.at[pl.ds(0, 128)], sem).wait()
    pltpu.make_async_copy(item_emb.at[ib], buf.at[pl.ds(128, 72)], sem).wait()

  def acc_row(buf, r):
    def tokd(j, carry):
      a0, a1, a2, a3 = carry
      t = j * 2
      a0 = a0 + buf[t, pl.ds(0, 16)] + buf[t + 1, pl.ds(0, 16)]
      a1 = a1 + buf[t, pl.ds(16, 16)] + buf[t + 1, pl.ds(16, 16)]
      a2 = a2 + buf[t, pl.ds(32, 16)] + buf[t + 1, pl.ds(32, 16)]
      a3 = a3 + buf[t, pl.ds(48, 16)] + buf[t + 1, pl.ds(48, 16)]
      return (a0, a1, a2, a3)
    a0, a1, a2, a3 = lax.fori_loop(0, L // 2, tokd, (zf, zf, zf, zf))
    acc[r, pl.ds(0, 16)] = a0
    acc[r, pl.ds(16, 16)] = a1
    acc[r, pl.ds(32, 16)] = a2
    acc[r, pl.ds(48, 16)] = a3

  stage_idx(base, ia0, ib0)
  start_gather(ia0, ib0, rows0, sem0)

  def pair(j, _):
    r0 = base + j * 2
    stage_idx(r0 + 1, ia1, ib1)
    start_gather(ia1, ib1, rows1, sem1)
    wait_gather(ia0, ib0, rows0, sem0)
    acc_row(rows0, j * 2)

    @pl.when(j + 1 < RPW // 2)
    def _():
      stage_idx(r0 + 2, ia0, ib0)
      start_gather(ia0, ib0, rows0, sem0)
    wait_gather(ia1, ib1, rows1, sem1)
    acc_row(rows1, j * 2 + 1)
    return None
  lax.fori_loop(0, RPW // 2, pair, None)

  pltpu.sync_copy(acc, o_sum_item.at[pl.ds(base, RPW)])


def _sc_call(item_seq, cat_seq, inter_seq, dur_seq, item_id, video_cat, age,
             gender, item_emb, cat_emb, age_emb, gender_emb):
  f32 = jnp.float32
  mesh = plsc.VectorSubcoreMesh(core_axis_name="c", subcore_axis_name="s")
  out_type = [
      jax.ShapeDtypeStruct((B, D), f32),
      jax.ShapeDtypeStruct((B, NCAT_P), f32),
      jax.ShapeDtypeStruct((B, NSMALL), f32),
      jax.ShapeDtypeStruct((B, NSMALL), f32),
      jax.ShapeDtypeStruct((B, 128), f32),
      jax.ShapeDtypeStruct((B, 128), f32),
      jax.ShapeDtypeStruct((B, 128), f32),
      jax.ShapeDtypeStruct((B, 128), f32),
  ]
  scratch = [
      pltpu.VMEM((128,), jnp.int32),
      pltpu.VMEM((72,), jnp.int32),
      pltpu.VMEM((128,), jnp.int32),
      pltpu.VMEM((72,), jnp.int32),
      pltpu.VMEM((L, 128), f32),
      pltpu.VMEM((L, 128), f32),
      pltpu.VMEM((RPW, D), f32),
      pltpu.VMEM((16 * L,), jnp.int32),
      pltpu.VMEM((16 * L,), jnp.int32),
      pltpu.VMEM((16 * L,), jnp.int32),
      pltpu.VMEM((16 * L,), jnp.int32),
      pltpu.VMEM((16, NCAT_P), f32),
      pltpu.VMEM((16, NSMALL), f32),
      pltpu.VMEM((16, NSMALL), f32),
      pltpu.SemaphoreType.DMA,
      pltpu.SemaphoreType.DMA,
  ]
  fn = pl.kernel(_sc_body, out_type=out_type, mesh=mesh,
                 scratch_types=scratch,
                 compiler_params=pltpu.CompilerParams(
                     use_tc_tiling_on_sc=True,
                     needs_layout_passes=False))
  return fn(item_seq, cat_seq, inter_seq, dur_seq, item_id, video_cat, age,
            gender, item_emb, cat_emb, age_emb, gender_emb)


def _tc_cat_body(cat_cnt, cat_t, o_catsum, o_c):
  cc = cat_cnt[...]
  o_catsum[...] = jnp.dot(cc, cat_t[...], precision=jax.lax.Precision.HIGHEST)
  o_c[...] = jnp.sum(cc, axis=1, keepdims=True)


def _tc_body(sum_item, catsum, c_in, inter_cnt, dur_cnt, ei, ec, eage, egen,
             ua, pop, e0, inter_t, dur_t,
             w_it, w_ct, w_in, w_du, w_ua, w_ag, w_ge, ub0, ug0, ube0,
             uW1, ub1, ug1, ube1, uW2, ub2,
             iw_ei, iw_ec, iw_pop, ib0, ig0, ibe0,
             iW1, ib1, ig1, ibe1, iW2, ib2,
             o_u, o_i):
  hi = jax.lax.Precision.HIGHEST

  def dot(a, b):
    return jnp.dot(a, b, precision=hi)

  def bn(x, g, b):
    m = jnp.mean(x, axis=0, keepdims=True)
    v = jnp.mean((x - m) * (x - m), axis=0, keepdims=True)
    return g[...] * (x - m) / jnp.sqrt(v + 1e-5) + b[...]

  def l2n(x):
    n = jnp.sqrt(jnp.sum(x * x, axis=1, keepdims=True))
    return x / jnp.maximum(n, 1e-12)

  c = c_in[...]
  denom = jnp.maximum(c, 1e-9)
  n0 = jnp.float32(L) - c
  p_item = (sum_item[...] - n0 * e0[...]) / denom
  p_cat = catsum[...] / denom
  p_inter = dot(inter_cnt[...], inter_t[...]) / denom
  p_dur = dot(dur_cnt[...], dur_t[...]) / denom

  h = (dot(p_item, w_it[...]) + dot(p_cat, w_ct[...]) +
       dot(p_inter, w_in[...]) + dot(p_dur, w_du[...]) +
       ua[...] * w_ua[...] + dot(eage[...], w_ag[...]) +
       dot(egen[...], w_ge[...]) + ub0[...])
  h = jnp.maximum(bn(h, ug0[...], ube0[...]), 0.0)
  h = jnp.maximum(bn(dot(h, uW1[...]) + ub1[...], ug1[...], ube1[...]), 0.0)
  u = dot(h, uW2[...]) + ub2[...]

  h = (dot(ei[...], iw_ei[...]) + dot(ec[...], iw_ec[...]) +
       pop[...] * iw_pop[...] + ib0[...])
  h = jnp.maximum(bn(h, ig0[...], ibe0[...]), 0.0)
  h = jnp.maximum(bn(dot(h, iW1[...]) + ib1[...], ig1[...], ibe1[...]), 0.0)
  iv = dot(h, iW2[...]) + ib2[...]

  o_u[...] = l2n(u)
  o_i[...] = l2n(iv)


def kernel(item_id_seq, video_category_seq, inter_type_seq,
           duration_bucket_seq, user_activity_norm, age, gender, item_id,
           video_category, item_pop_norm, item_emb, cat_emb, inter_emb,
           dur_emb, age_emb, gender_emb, uW0, ub0, ug0, ube0, uW1, ub1, ug1,
           ube1, uW2, ub2, iW0, ib0, ig0, ibe0, iW1, ib1, ig1, ibe1, iW2,
           ib2):
  i32 = jnp.int32
  f32 = jnp.float32
  item_seq = item_id_seq.astype(i32).reshape(-1)
  cat_seq = video_category_seq.astype(i32).reshape(-1)
  inter_seq = inter_type_seq.astype(i32).reshape(-1)
  dur_seq = duration_bucket_seq.astype(i32).reshape(-1)

  item_pad = jnp.transpose(jnp.pad(jnp.transpose(item_emb),
                                   ((0, 128 - D), (0, 0))))
  cat_pad = jnp.pad(cat_emb, ((0, 0), (0, 128 - D)))
  age_pad = jnp.pad(age_emb, ((0, 0), (0, 128 - D // 2)))
  gen_pad = jnp.pad(gender_emb, ((0, 0), (0, 128 - D // 2)))

  (sum_item, cat_cnt, inter_cnt, dur_cnt, ei128, ec128, eage128,
   egen128) = _sc_call(
      item_seq, cat_seq, inter_seq, dur_seq, item_id.astype(i32),
      video_category.astype(i32), age.astype(i32), gender.astype(i32),
      item_pad, cat_pad, age_pad, gen_pad)
  ei = lax.slice(ei128, (0, 0), (B, D))
  ec = lax.slice(ec128, (0, 0), (B, D))
  eage = lax.slice(eage128, (0, 0), (B, D // 2))
  egen = lax.slice(egen128, (0, 0), (B, D // 2))

  e0 = lax.slice(item_emb, (0, 0), (1, D))
  cat_t = jnp.pad(cat_emb, ((0, NCAT_P - cat_emb.shape[0]), (0, 0)))
  inter_t = jnp.pad(inter_emb, ((0, NSMALL - inter_emb.shape[0]), (0, 0)))
  dur_t = jnp.pad(dur_emb, ((0, NSMALL - dur_emb.shape[0]), (0, 0)))

  w_it = lax.slice(uW0, (0, 0), (D, 128))
  w_ct = lax.slice(uW0, (D, 0), (2 * D, 128))
  w_in = lax.slice(uW0, (2 * D, 0), (3 * D, 128))
  w_du = lax.slice(uW0, (3 * D, 0), (4 * D, 128))
  w_ua = lax.slice(uW0, (4 * D, 0), (4 * D + 1, 128))
  w_ag = lax.slice(uW0, (4 * D + 1, 0), (4 * D + 1 + D // 2, 128))
  w_ge = lax.slice(uW0, (4 * D + 1 + D // 2, 0), (4 * D + 1 + D, 128))
  iw_ei = lax.slice(iW0, (0, 0), (D, 128))
  iw_ec = lax.slice(iW0, (D, 0), (2 * D, 128))
  iw_pop = lax.slice(iW0, (2 * D, 0), (2 * D + 1, 128))

  def r2(v):
    return v.reshape(1, -1)

  nblk = 8
  rows = B // nblk
  catsum, c = pl.pallas_call(
      _tc_cat_body,
      grid=(nblk,),
      in_specs=[pl.BlockSpec((rows, NCAT_P), lambda i: (i, 0)),
                pl.BlockSpec((NCAT_P, D), lambda i: (0, 0))],
      out_specs=[pl.BlockSpec((rows, D), lambda i: (i, 0)),
                 pl.BlockSpec((rows, 1), lambda i: (i, 0))],
      out_shape=[jax.ShapeDtypeStruct((B, D), f32),
                 jax.ShapeDtypeStruct((B, 1), f32)])(cat_cnt, cat_t)

  out_shape = (jax.ShapeDtypeStruct((B, D), f32),
               jax.ShapeDtypeStruct((B, D), f32))
  return pl.pallas_call(_tc_body, out_shape=out_shape)(
      sum_item, catsum, c, inter_cnt, dur_cnt, ei, ec, eage, egen,
      user_activity_norm, item_pop_norm, e0, inter_t, dur_t,
      w_it, w_ct, w_in, w_du, w_ua, w_ag, w_ge,
      r2(ub0), r2(ug0), r2(ube0), uW1, r2(ub1), r2(ug1), r2(ube1),
      uW2, r2(ub2),
      iw_ei, iw_ec, iw_pop, r2(ib0), r2(ig0), r2(ibe0),
      iW1, r2(ib1), r2(ig1), r2(ibe1), iW2, r2(ib2))


# speedup vs baseline: 1.1818x; 1.1153x over previous
import functools

import jax
import jax.numpy as jnp
from jax import lax
from jax.experimental import pallas as pl
from jax.experimental.pallas import tpu as pltpu
from jax.experimental.pallas import tpu_sc as plsc

B = 4096
L = 200
D = 64
NCAT_P = 1008
NSMALL = 16
NW = 32
RPW = B // NW
NG = RPW // 16


def _sc_hist_body(item_seq, cat_seq, inter_seq, dur_seq, video_cat, age,
                  gender, cat_emb, age_emb, gender_emb,
                  o_cat_cnt, o_inter_cnt, o_dur_cnt, o_ec, o_eage, o_egen,
                  ia0, gbuf, slab_it, slab_ct, slab_in, slab_du, hist_c,
                  hist_i, hist_d, sem0):
  nc = 2
  wid = lax.axis_index("s") * nc + lax.axis_index("c")
  base = wid * RPW

  zf = jnp.zeros((16,), jnp.float32)
  ones16 = jnp.ones((16,), jnp.float32)
  row_iota = lax.iota(jnp.int32, 16)

  def single_gather(idx_src, table, out):
    pltpu.sync_copy(idx_src.at[pl.ds(base, RPW)], ia0)
    pltpu.async_copy(table.at[ia0], gbuf, sem0).wait()
    pltpu.sync_copy(gbuf, out.at[pl.ds(base, RPW)])

  single_gather(video_cat, cat_emb, o_ec)
  single_gather(age, age_emb, o_eage)
  single_gather(gender, gender_emb, o_egen)

  def zero_hist(r, _):
    def zcol(j, _):
      hist_c[r, pl.ds(j * 16, 16)] = zf
      return None
    lax.fori_loop(0, NCAT_P // 16, zcol, None)
    hist_i[r, :] = zf
    hist_d[r, :] = zf
    return None
  lax.fori_loop(0, 16, zero_hist, None)

  row_off = row_iota * L
  def group(g, _):
    r0 = base + g * 16
    pltpu.sync_copy(item_seq.at[pl.ds(r0 * L, 16 * L)], slab_it)
    pltpu.sync_copy(cat_seq.at[pl.ds(r0 * L, 16 * L)], slab_ct)
    pltpu.sync_copy(inter_seq.at[pl.ds(r0 * L, 16 * L)], slab_in)
    pltpu.sync_copy(dur_seq.at[pl.ds(r0 * L, 16 * L)], slab_du)

    def tok(t, _):
      tv = row_off + t
      it_v = plsc.load_gather(slab_it, [tv])
      m = it_v > 0
      ct_v = plsc.load_gather(slab_ct, [tv])
      in_v = plsc.load_gather(slab_in, [tv])
      du_v = plsc.load_gather(slab_du, [tv])
      plsc.addupdate_scatter(hist_c, [row_iota, ct_v], ones16, mask=m)
      plsc.addupdate_scatter(hist_i, [row_iota, in_v], ones16, mask=m)
      plsc.addupdate_scatter(hist_d, [row_iota, du_v], ones16, mask=m)
      return None
    lax.fori_loop(0, L, tok, None)

    pltpu.sync_copy(hist_c, o_cat_cnt.at[pl.ds(r0, 16)])
    pltpu.sync_copy(hist_i, o_inter_cnt.at[pl.ds(r0, 16)])
    pltpu.sync_copy(hist_d, o_dur_cnt.at[pl.ds(r0, 16)])
    lax.fori_loop(0, 16, zero_hist, None)
    return None
  lax.fori_loop(0, NG, group, None)


def _sc_item_body(item_seq, item_id, item_emb,
                  o_sum_item, o_ei,
                  ia0, ib0, ia1, ib1, rows0, rows1, acc, sem0, sem1):
  nc = 2
  wid = lax.axis_index("s") * nc + lax.axis_index("c")
  base = wid * RPW

  zf = jnp.zeros((16,), jnp.float32)

  pltpu.sync_copy(item_id.at[pl.ds(base, RPW)], ia0)
  pltpu.async_copy(item_emb.at[ia0], rows0.at[pl.ds(0, RPW)], sem0).wait()
  pltpu.sync_copy(rows0.at[pl.ds(0, RPW)], o_ei.at[pl.ds(base, RPW)])

  def stage_idx(r, ia, ib):
    pltpu.sync_copy(item_seq.at[pl.ds(r * L, 128)], ia)
    pltpu.sync_copy(item_seq.at[pl.ds(r * L + 128, 72)], ib)

  def start_gather(ia, ib, buf, sem):
    pltpu.async_copy(item_emb.at[ia], buf.at[pl.ds(0, 128)], sem)
    pltpu.async_copy(item_emb.at[ib], buf.at[pl.ds(128, 72)], sem)

  def wait_gather(ia, ib, buf, sem):
    pltpu.make_async_copy(item_emb.at[ia], buf.at[pl.ds(0, 128)], sem).wait()
    pltpu.make_async_copy(item_emb.at[ib], buf.at[pl.ds(128, 72)], sem).wait()

  def acc_row(buf, r):
    def tokd(j, carry):
      a0, a1, a2, a3 = carry
      t = j * 2
      a0 = a0 + buf[t, pl.ds(0, 16)] + buf[t + 1, pl.ds(0, 16)]
      a1 = a1 + buf[t, pl.ds(16, 16)] + buf[t + 1, pl.ds(16, 16)]
      a2 = a2 + buf[t, pl.ds(32, 16)] + buf[t + 1, pl.ds(32, 16)]
      a3 = a3 + buf[t, pl.ds(48, 16)] + buf[t + 1, pl.ds(48, 16)]
      return (a0, a1, a2, a3)
    a0, a1, a2, a3 = lax.fori_loop(0, L // 2, tokd, (zf, zf, zf, zf))
    acc[r, pl.ds(0, 16)] = a0
    acc[r, pl.ds(16, 16)] = a1
    acc[r, pl.ds(32, 16)] = a2
    acc[r, pl.ds(48, 16)] = a3

  stage_idx(base, ia0, ib0)
  start_gather(ia0, ib0, rows0, sem0)

  def pair(j, _):
    r0 = base + j * 2
    stage_idx(r0 + 1, ia1, ib1)
    start_gather(ia1, ib1, rows1, sem1)
    wait_gather(ia0, ib0, rows0, sem0)
    acc_row(rows0, j * 2)

    @pl.when(j + 1 < RPW // 2)
    def _():
      stage_idx(r0 + 2, ia0, ib0)
      start_gather(ia0, ib0, rows0, sem0)
    wait_gather(ia1, ib1, rows1, sem1)
    acc_row(rows1, j * 2 + 1)
    return None
  lax.fori_loop(0, RPW // 2, pair, None)

  pltpu.sync_copy(acc, o_sum_item.at[pl.ds(base, RPW)])


_SC_PARAMS = dict(compiler_params=pltpu.CompilerParams(
    use_tc_tiling_on_sc=True, needs_layout_passes=False))


def _sc_hist_call(item_seq, cat_seq, inter_seq, dur_seq, video_cat, age,
                  gender, cat_emb, age_emb, gender_emb):
  f32 = jnp.float32
  mesh = plsc.VectorSubcoreMesh(core_axis_name="c", subcore_axis_name="s")
  out_type = [
      jax.ShapeDtypeStruct((B, NCAT_P), f32),
      jax.ShapeDtypeStruct((B, NSMALL), f32),
      jax.ShapeDtypeStruct((B, NSMALL), f32),
      jax.ShapeDtypeStruct((B, 128), f32),
      jax.ShapeDtypeStruct((B, 128), f32),
      jax.ShapeDtypeStruct((B, 128), f32),
  ]
  scratch = [
      pltpu.VMEM((128,), jnp.int32),
      pltpu.VMEM((RPW, 128), f32),
      pltpu.VMEM((16 * L,), jnp.int32),
      pltpu.VMEM((16 * L,), jnp.int32),
      pltpu.VMEM((16 * L,), jnp.int32),
      pltpu.VMEM((16 * L,), jnp.int32),
      pltpu.VMEM((16, NCAT_P), f32),
      pltpu.VMEM((16, NSMALL), f32),
      pltpu.VMEM((16, NSMALL), f32),
      pltpu.SemaphoreType.DMA,
  ]
  fn = pl.kernel(_sc_hist_body, out_type=out_type, mesh=mesh,
                 scratch_types=scratch, **_SC_PARAMS)
  return fn(item_seq, cat_seq, inter_seq, dur_seq, video_cat, age, gender,
            cat_emb, age_emb, gender_emb)


def _sc_item_call(item_seq, item_id, item_emb):
  f32 = jnp.float32
  mesh = plsc.VectorSubcoreMesh(core_axis_name="c", subcore_axis_name="s")
  out_type = [
      jax.ShapeDtypeStruct((B, D), f32),
      jax.ShapeDtypeStruct((B, 128), f32),
  ]
  scratch = [
      pltpu.VMEM((128,), jnp.int32),
      pltpu.VMEM((72,), jnp.int32),
      pltpu.VMEM((128,), jnp.int32),
      pltpu.VMEM((72,), jnp.int32),
      pltpu.VMEM((L, 128), f32),
      pltpu.VMEM((L, 128), f32),
      pltpu.VMEM((RPW, D), f32),
      pltpu.SemaphoreType.DMA,
      pltpu.SemaphoreType.DMA,
  ]
  fn = pl.kernel(_sc_item_body, out_type=out_type, mesh=mesh,
                 scratch_types=scratch, **_SC_PARAMS)
  return fn(item_seq, item_id, item_emb)


def _tc_cat_body(cat_cnt, cat_t, o_catsum, o_c):
  cc = cat_cnt[...]
  o_catsum[...] = jnp.dot(cc, cat_t[...], precision=jax.lax.Precision.HIGHEST)
  o_c[...] = jnp.sum(cc, axis=1, keepdims=True)


def _tc_body(sum_item, catsum, c_in, inter_cnt, dur_cnt, ei, ec, eage, egen,
             ua, pop, e0, inter_t, dur_t,
             w_it, w_ct, w_in, w_du, w_ua, w_ag, w_ge, ub0, ug0, ube0,
             uW1, ub1, ug1, ube1, uW2, ub2,
             iw_ei, iw_ec, iw_pop, ib0, ig0, ibe0,
             iW1, ib1, ig1, ibe1, iW2, ib2,
             o_u, o_i):
  hi = jax.lax.Precision.HIGHEST

  def dot(a, b):
    return jnp.dot(a, b, precision=hi)

  def bn(x, g, b):
    m = jnp.mean(x, axis=0, keepdims=True)
    v = jnp.mean((x - m) * (x - m), axis=0, keepdims=True)
    return g[...] * (x - m) / jnp.sqrt(v + 1e-5) + b[...]

  def l2n(x):
    n = jnp.sqrt(jnp.sum(x * x, axis=1, keepdims=True))
    return x / jnp.maximum(n, 1e-12)

  c = c_in[...]
  denom = jnp.maximum(c, 1e-9)
  n0 = jnp.float32(L) - c
  p_item = (sum_item[...] - n0 * e0[...]) / denom
  p_cat = catsum[...] / denom
  p_inter = dot(inter_cnt[...], inter_t[...]) / denom
  p_dur = dot(dur_cnt[...], dur_t[...]) / denom

  h = (dot(p_item, w_it[...]) + dot(p_cat, w_ct[...]) +
       dot(p_inter, w_in[...]) + dot(p_dur, w_du[...]) +
       ua[...] * w_ua[...] + dot(eage[...], w_ag[...]) +
       dot(egen[...], w_ge[...]) + ub0[...])
  h = jnp.maximum(bn(h, ug0[...], ube0[...]), 0.0)
  h = jnp.maximum(bn(dot(h, uW1[...]) + ub1[...], ug1[...], ube1[...]), 0.0)
  u = dot(h, uW2[...]) + ub2[...]

  h = (dot(ei[...], iw_ei[...]) + dot(ec[...], iw_ec[...]) +
       pop[...] * iw_pop[...] + ib0[...])
  h = jnp.maximum(bn(h, ig0[...], ibe0[...]), 0.0)
  h = jnp.maximum(bn(dot(h, iW1[...]) + ib1[...], ig1[...], ibe1[...]), 0.0)
  iv = dot(h, iW2[...]) + ib2[...]

  o_u[...] = l2n(u)
  o_i[...] = l2n(iv)


def kernel(item_id_seq, video_category_seq, inter_type_seq,
           duration_bucket_seq, user_activity_norm, age, gender, item_id,
           video_category, item_pop_norm, item_emb, cat_emb, inter_emb,
           dur_emb, age_emb, gender_emb, uW0, ub0, ug0, ube0, uW1, ub1, ug1,
           ube1, uW2, ub2, iW0, ib0, ig0, ibe0, iW1, ib1, ig1, ibe1, iW2,
           ib2):
  i32 = jnp.int32
  f32 = jnp.float32
  item_seq = item_id_seq.astype(i32).reshape(-1)
  cat_seq = video_category_seq.astype(i32).reshape(-1)
  inter_seq = inter_type_seq.astype(i32).reshape(-1)
  dur_seq = duration_bucket_seq.astype(i32).reshape(-1)

  item_pad = jnp.transpose(jnp.pad(jnp.transpose(item_emb),
                                   ((0, 128 - D), (0, 0))))
  cat_pad = jnp.pad(cat_emb, ((0, 0), (0, 128 - D)))
  age_pad = jnp.pad(age_emb, ((0, 0), (0, 128 - D // 2)))
  gen_pad = jnp.pad(gender_emb, ((0, 0), (0, 128 - D // 2)))

  (cat_cnt, inter_cnt, dur_cnt, ec128, eage128, egen128) = _sc_hist_call(
      item_seq, cat_seq, inter_seq, dur_seq, video_category.astype(i32),
      age.astype(i32), gender.astype(i32), cat_pad, age_pad, gen_pad)
  sum_item, ei128 = _sc_item_call(item_seq, item_id.astype(i32), item_pad)
  ei = lax.slice(ei128, (0, 0), (B, D))
  ec = lax.slice(ec128, (0, 0), (B, D))
  eage = lax.slice(eage128, (0, 0), (B, D // 2))
  egen = lax.slice(egen128, (0, 0), (B, D // 2))

  e0 = lax.slice(item_emb, (0, 0), (1, D))
  cat_t = jnp.pad(cat_emb, ((0, NCAT_P - cat_emb.shape[0]), (0, 0)))
  inter_t = jnp.pad(inter_emb, ((0, NSMALL - inter_emb.shape[0]), (0, 0)))
  dur_t = jnp.pad(dur_emb, ((0, NSMALL - dur_emb.shape[0]), (0, 0)))

  w_it = lax.slice(uW0, (0, 0), (D, 128))
  w_ct = lax.slice(uW0, (D, 0), (2 * D, 128))
  w_in = lax.slice(uW0, (2 * D, 0), (3 * D, 128))
  w_du = lax.slice(uW0, (3 * D, 0), (4 * D, 128))
  w_ua = lax.slice(uW0, (4 * D, 0), (4 * D + 1, 128))
  w_ag = lax.slice(uW0, (4 * D + 1, 0), (4 * D + 1 + D // 2, 128))
  w_ge = lax.slice(uW0, (4 * D + 1 + D // 2, 0), (4 * D + 1 + D, 128))
  iw_ei = lax.slice(iW0, (0, 0), (D, 128))
  iw_ec = lax.slice(iW0, (D, 0), (2 * D, 128))
  iw_pop = lax.slice(iW0, (2 * D, 0), (2 * D + 1, 128))

  def r2(v):
    return v.reshape(1, -1)

  nblk = 8
  rows = B // nblk
  catsum, c = pl.pallas_call(
      _tc_cat_body,
      grid=(nblk,),
      in_specs=[pl.BlockSpec((rows, NCAT_P), lambda i: (i, 0)),
                pl.BlockSpec((NCAT_P, D), lambda i: (0, 0))],
      out_specs=[pl.BlockSpec((rows, D), lambda i: (i, 0)),
                 pl.BlockSpec((rows, 1), lambda i: (i, 0))],
      out_shape=[jax.ShapeDtypeStruct((B, D), f32),
                 jax.ShapeDtypeStruct((B, 1), f32)])(cat_cnt, cat_t)

  out_shape = (jax.ShapeDtypeStruct((B, D), f32),
               jax.ShapeDtypeStruct((B, D), f32))
  return pl.pallas_call(_tc_body, out_shape=out_shape)(
      sum_item, catsum, c, inter_cnt, dur_cnt, ei, ec, eage, egen,
      user_activity_norm, item_pop_norm, e0, inter_t, dur_t,
      w_it, w_ct, w_in, w_du, w_ua, w_ag, w_ge,
      r2(ub0), r2(ug0), r2(ube0), uW1, r2(ub1), r2(ug1), r2(ube1),
      uW2, r2(ub2),
      iw_ei, iw_ec, iw_pop, r2(ib0), r2(ig0), r2(ibe0),
      iW1, r2(ib1), r2(ig1), r2(ibe1), iW2, r2(ib2))
